# baseline (speedup 1.0000x reference)
import functools

import jax
import jax.numpy as jnp
from jax import lax
from jax.experimental import pallas as pl
from jax.experimental.pallas import tpu as pltpu
from jax.experimental.pallas import tpu_sc as plsc

N = 10000
E = 160000
IN_DIM = 50
H = 8
C = 64
D = 512
OUT = 121
OUTP = 128

BN = 1000
NB = N // BN

NCORES = 2
NSUB = 16
CH = 400
STRIPE = 624
TAIL = N - NSUB * STRIPE
ZROWS = 156

_f32 = jnp.float32
_i32 = jnp.int32


def _stats_from_h(h, as_ref, ad_ref, heads, c):
    bn = h.shape[0]
    cols_s, cols_d = [], []
    for h2 in range(heads):
        hh = h[:, c * h2:c * h2 + c]
        cols_s.append(jnp.sum(hh * as_ref[h2:h2 + 1, :], axis=1, keepdims=True))
        cols_d.append(jnp.sum(hh * ad_ref[h2:h2 + 1, :], axis=1, keepdims=True))
    pad = [jnp.zeros((bn, 16 - heads), dtype=_f32)]
    return (jnp.concatenate(cols_s + pad, axis=1),
            jnp.concatenate(cols_d + pad, axis=1))


def _pre1_body(x_ref, W1_ref, as_ref, ad_ref, R1_ref,
               h_ref, sts_ref, std_ref, r_ref):
    xb = x_ref[...]
    h = jnp.dot(xb, W1_ref[...], preferred_element_type=_f32)
    h_ref[...] = h
    sts_ref[...], std_ref[...] = _stats_from_h(h, as_ref, ad_ref, H, C)
    r_ref[...] = jnp.dot(xb, R1_ref[...], preferred_element_type=_f32)


def _epilogue(acc_ref, s_ref, r_ref, b_ref, g_ref, be_ref):
    s = s_ref[0, :, :8] + s_ref[1, :, :8]
    rinv = 1.0 / (s + 1e-16)
    acc = acc_ref[...]
    v = jnp.concatenate([acc[h2] * rinv[:, h2:h2 + 1] for h2 in range(H)],
                        axis=1)
    v = v + b_ref[...] + r_ref[...]
    mu = jnp.mean(v, axis=1, keepdims=True)
    var = jnp.mean((v - mu) ** 2, axis=1, keepdims=True)
    y = (v - mu) * lax.rsqrt(var + 1e-5) * g_ref[...] + be_ref[...]
    return jnp.where(y > 0, y, jnp.exp(jnp.minimum(y, 0.0)) - 1.0)


def _mid_body(W_ref, as_ref, ad_ref, R_ref,
              acc_ref, s_ref, r_ref, b_ref, g_ref, be_ref,
              h_ref, sts_ref, std_ref, r2_ref, *, heads, c):
    y = _epilogue(acc_ref, s_ref, r_ref, b_ref, g_ref, be_ref)
    h = jnp.dot(y, W_ref[...], preferred_element_type=_f32)
    h_ref[...] = h
    sts_ref[...], std_ref[...] = _stats_from_h(h, as_ref, ad_ref, heads, c)
    r2_ref[...] = jnp.dot(y, R_ref[...], preferred_element_type=_f32)


def _post3_body(acc_ref, s_ref, r_ref, b_ref, o_ref):
    o = jnp.concatenate([acc_ref[0] + acc_ref[2],
                         acc_ref[1] + acc_ref[3]], axis=1)
    stot = s_ref[0, :, 0:1] + s_ref[1, :, 0:1]
    o_ref[...] = o / (stot + 1e-16) + b_ref[...] + r_ref[...]


def _pre1(x, W1, as1, ad1, R1):
    full = lambda s: pl.BlockSpec(s, lambda i: (0,) * len(s))
    return pl.pallas_call(
        _pre1_body,
        grid=(NB,),
        in_specs=[pl.BlockSpec((BN, IN_DIM), lambda i: (i, 0)),
                  full((IN_DIM, D)), full((H, C)), full((H, C)),
                  full((IN_DIM, D))],
        out_specs=[pl.BlockSpec((BN, D), lambda i: (i, 0)),
                   pl.BlockSpec((BN, 16), lambda i: (i, 0)),
                   pl.BlockSpec((BN, 16), lambda i: (i, 0)),
                   pl.BlockSpec((BN, D), lambda i: (i, 0))],
        out_shape=[jax.ShapeDtypeStruct((N, D), _f32),
                   jax.ShapeDtypeStruct((N, 16), _f32),
                   jax.ShapeDtypeStruct((N, 16), _f32),
                   jax.ShapeDtypeStruct((N, D), _f32)],
    )(x, W1, as1, ad1, R1)


def _mid(acc, s, r, b2d, g2d, be2d, W, asx, adx, R, heads, c, dout):
    full = lambda sh: pl.BlockSpec(sh, lambda i: (0,) * len(sh))
    kdim = W.shape[0]
    return pl.pallas_call(
        functools.partial(_mid_body, heads=heads, c=c),
        grid=(NB,),
        in_specs=[full((kdim, dout)), full((heads, c)), full((heads, c)),
                  full((kdim, dout)),
                  pl.BlockSpec((H, BN, C), lambda i: (0, i, 0)),
                  pl.BlockSpec((2, BN, 16), lambda i: (0, i, 0)),
                  pl.BlockSpec((BN, D), lambda i: (i, 0)),
                  full((1, D)), full((1, D)), full((1, D))],
        out_specs=[pl.BlockSpec((BN, dout), lambda i: (i, 0)),
                   pl.BlockSpec((BN, 16), lambda i: (i, 0)),
                   pl.BlockSpec((BN, 16), lambda i: (i, 0)),
                   pl.BlockSpec((BN, dout), lambda i: (i, 0))],
        out_shape=[jax.ShapeDtypeStruct((N, dout), _f32),
                   jax.ShapeDtypeStruct((N, 16), _f32),
                   jax.ShapeDtypeStruct((N, 16), _f32),
                   jax.ShapeDtypeStruct((N, dout), _f32)],
    )(W, asx, adx, R, acc, s, r, b2d, g2d, be2d)


def _post3(acc3, s3, r3, b3p2d):
    full = lambda sh: pl.BlockSpec(sh, lambda i: (0,) * len(sh))
    return pl.pallas_call(
        _post3_body,
        grid=(NB,),
        in_specs=[pl.BlockSpec((4, BN, C), lambda i: (0, i, 0)),
                  pl.BlockSpec((2, BN, 16), lambda i: (0, i, 0)),
                  pl.BlockSpec((BN, OUTP), lambda i: (i, 0)),
                  full((1, OUTP))],
        out_specs=pl.BlockSpec((BN, OUTP), lambda i: (i, 0)),
        out_shape=jax.ShapeDtypeStruct((N, OUTP), _f32),
    )(acc3, s3, r3, b3p2d)


_COL8 = None


def _zero_vmem(ref, nrows, ncol):
    def row(i, _):
        for q in range(ncol // 16):
            ref[i, pl.ds(16 * q, 16)] = jnp.zeros((16,), _f32)
        return _
    lax.fori_loop(0, nrows, row, None)


def _for_stripe(sid, fn):
    fn(sid * STRIPE, STRIPE)

    @pl.when(sid == NSUB - 1)
    def _():
        fn(NSUB * STRIPE, TAIL)


def _zero_stripe(zb, acc, base, size):
    for k in range(size // ZROWS):
        pltpu.sync_copy(zb, acc.at[pl.ds(base + k * ZROWS, ZROWS)])
    rem = size % ZROWS
    if rem:
        pltpu.sync_copy(zb.at[pl.ds(0, rem)],
                        acc.at[pl.ds(base + size - rem, rem)])


def _compute_ex(sst, dstst, exb, n_edges):
    @plsc.parallel_loop(0, n_edges, unroll=8)
    def body(e):
        t = sst[e] + dstst[e]
        exb[e] = jnp.exp(jnp.maximum(t, 0.2 * t))


def _splat16(vec, lane):
    return vec.at[jnp.full((16,), lane, _i32)].get(mode="promise_in_bounds")


def _scale_rows(rows, exb, head, n_edges, nq):
    @plsc.parallel_loop(0, n_edges, unroll=8)
    def body(e):
        exs = _splat16(exb[e], head)
        for q in range(nq):
            rows[e, pl.ds(16 * q, 16)] = rows[e, pl.ds(16 * q, 16)] * exs


_NCHUNK12 = E // NSUB // CH


def _sc12_body(src_hbm, dst_hbm, hview, stats_s, stats_d,
               out_hm, s_out, ex_hbm,
               src_v, dst_v, idx_v, sst, dstst, exb, rows,
               zb, zbs, acc, s_acc, sem1, sem2, sem3):
    cid = lax.axis_index("c")
    sid = lax.axis_index("s")
    base0 = sid * (E // NSUB)

    for p in range(4):
        head = 4 * cid + p
        _zero_vmem(zb, ZROWS, C)
        _for_stripe(sid, functools.partial(_zero_stripe, zb, acc))
        if p == 0:
            _zero_vmem(zbs, ZROWS, 16)
            _for_stripe(sid, functools.partial(_zero_stripe, zbs, s_acc))
        plsc.subcore_barrier()

        def chunk(kc, _):
            eb = base0 + kc * CH
            pltpu.sync_copy(src_hbm.at[pl.ds(eb, CH)], src_v)
            pltpu.sync_copy(dst_hbm.at[pl.ds(eb, CH)], dst_v)
            if p == 0:
                ga = pltpu.async_copy(stats_s.at[src_v], sst, sem1)
                gb = pltpu.async_copy(stats_d.at[dst_v], dstst, sem2)
                ga.wait()
                gb.wait()
                _compute_ex(sst, dstst, exb, CH)
                pltpu.sync_copy(exb, ex_hbm.at[pl.ds(eb, CH)])
                mine = jnp.where(cid == 0, kc < 13, kc >= 13)

                @pl.when(mine)
                def _():
                    pltpu.sync_copy(exb, s_acc.at[dst_v], add=True)
            else:
                pltpu.sync_copy(ex_hbm.at[pl.ds(eb, CH)], exb)
            for i in range(CH // 16):
                sl = pl.ds(16 * i, 16)
                idx_v[sl] = src_v[sl] * 8 + head
            pltpu.async_copy(hview.at[idx_v], rows, sem3).wait()
            _scale_rows(rows, exb, head, CH, C // 16)
            pltpu.sync_copy(rows, acc.at[dst_v], add=True)
            return _

        lax.fori_loop(0, _NCHUNK12, chunk, None)
        plsc.subcore_barrier()

        def cp(base, size):
            sl = pl.ds(base, size)
            pltpu.sync_copy(acc.at[sl], out_hm.at[head, sl])
            if p == 0:
                pltpu.sync_copy(s_acc.at[sl], s_out.at[cid, sl])
        _for_stripe(sid, cp)


def _sc3_body(src_hbm, dst_hbm, hview, stats_s, stats_d,
              out_p, s_out, ex_hbm,
              src_v, dst_v, idx_v, sst, dstst, exb, rows,
              zb, zbs, acc, s_acc, sem1, sem2, sem3):
    cid = lax.axis_index("c")
    sid = lax.axis_index("s")
    base0 = sid * (E // NSUB)

    for q in range(2):
        _zero_vmem(zb, ZROWS, C)
        _for_stripe(sid, functools.partial(_zero_stripe, zb, acc))
        if q == 0:
            _zero_vmem(zbs, ZROWS, 16)
            _for_stripe(sid, functools.partial(_zero_stripe, zbs, s_acc))
        plsc.subcore_barrier()

        def chunk(kc, _):
            mine = jnp.where(cid == 0, kc < 13, kc >= 13)

            @pl.when(mine)
            def _run():
                eb = base0 + kc * CH
                pltpu.sync_copy(src_hbm.at[pl.ds(eb, CH)], src_v)
                pltpu.sync_copy(dst_hbm.at[pl.ds(eb, CH)], dst_v)
                if q == 0:
                    ga = pltpu.async_copy(stats_s.at[src_v], sst, sem1)
                    gb = pltpu.async_copy(stats_d.at[dst_v], dstst, sem2)
                    ga.wait()
                    gb.wait()
                    _compute_ex(sst, dstst, exb, CH)
                    pltpu.sync_copy(exb, ex_hbm.at[pl.ds(eb, CH)])
                    pltpu.sync_copy(exb, s_acc.at[dst_v], add=True)
                else:
                    pltpu.sync_copy(ex_hbm.at[pl.ds(eb, CH)], exb)
                for i in range(CH // 16):
                    sl = pl.ds(16 * i, 16)
                    idx_v[sl] = src_v[sl] * 2 + q
                pltpu.async_copy(hview.at[idx_v], rows, sem3).wait()
                _scale_rows(rows, exb, 0, CH, C // 16)
                pltpu.sync_copy(rows, acc.at[dst_v], add=True)
            return _

        lax.fori_loop(0, _NCHUNK12, chunk, None)
        plsc.subcore_barrier()

        def cp(base, size):
            sl = pl.ds(base, size)
            pltpu.sync_copy(acc.at[sl], out_p.at[2 * cid + q, sl])
            if q == 0:
                pltpu.sync_copy(s_acc.at[sl], s_out.at[cid, sl])
        _for_stripe(sid, cp)


@functools.cache
def _sc_kernels():
    mesh = plsc.VectorSubcoreMesh(core_axis_name="c", subcore_axis_name="s",
                                  num_cores=NCORES, num_subcores=NSUB)
    params = pltpu.CompilerParams(use_tc_tiling_on_sc=False)
    sc12 = pl.kernel(
        _sc12_body,
        out_type=[jax.ShapeDtypeStruct((H, N, C), _f32),
                  jax.ShapeDtypeStruct((2, N, 16), _f32),
                  jax.ShapeDtypeStruct((E, 16), _f32)],
        mesh=mesh,
        scratch_types=[
            pltpu.VMEM((CH,), _i32), pltpu.VMEM((CH,), _i32),
            pltpu.VMEM((CH,), _i32),
            pltpu.VMEM((CH, 16), _f32), pltpu.VMEM((CH, 16), _f32),
            pltpu.VMEM((CH, 16), _f32), pltpu.VMEM((CH, C), _f32),
            pltpu.VMEM((ZROWS, C), _f32), pltpu.VMEM((ZROWS, 16), _f32),
            pltpu.VMEM_SHARED((N, C), _f32), pltpu.VMEM_SHARED((N, 16), _f32),
            pltpu.SemaphoreType.DMA, pltpu.SemaphoreType.DMA,
            pltpu.SemaphoreType.DMA,
        ],
        compiler_params=params,
    )

    sc3 = pl.kernel(
        _sc3_body,
        out_type=[jax.ShapeDtypeStruct((4, N, C), _f32),
                  jax.ShapeDtypeStruct((2, N, 16), _f32),
                  jax.ShapeDtypeStruct((E, 16), _f32)],
        mesh=mesh,
        scratch_types=[
            pltpu.VMEM((CH,), _i32), pltpu.VMEM((CH,), _i32),
            pltpu.VMEM((CH,), _i32),
            pltpu.VMEM((CH, 16), _f32), pltpu.VMEM((CH, 16), _f32),
            pltpu.VMEM((CH, 16), _f32), pltpu.VMEM((CH, C), _f32),
            pltpu.VMEM((ZROWS, C), _f32), pltpu.VMEM((ZROWS, 16), _f32),
            pltpu.VMEM_SHARED((N, C), _f32), pltpu.VMEM_SHARED((N, 16), _f32),
            pltpu.SemaphoreType.DMA, pltpu.SemaphoreType.DMA,
            pltpu.SemaphoreType.DMA,
        ],
        compiler_params=params,
    )
    return sc12, sc3


def kernel(x, edge_index, W1, att_src1, att_dst1, b1, R1, g1, beta1,
           W2, att_src2, att_dst2, b2, R2, g2, beta2,
           W3, att_src3, att_dst3, b3, R3):
    src = edge_index[0].astype(_i32)
    dst = edge_index[1].astype(_i32)

    b1r, g1r, be1r = b1.reshape(1, D), g1.reshape(1, D), beta1.reshape(1, D)
    b2r, g2r, be2r = b2.reshape(1, D), g2.reshape(1, D), beta2.reshape(1, D)
    W3p = jnp.pad(W3, ((0, 0), (0, OUTP - OUT)))
    R3p = jnp.pad(R3, ((0, 0), (0, OUTP - OUT)))
    as3p = jnp.pad(att_src3, ((0, 0), (0, OUTP - OUT)))
    ad3p = jnp.pad(att_dst3, ((0, 0), (0, OUTP - OUT)))
    b3p = jnp.pad(b3, (0, OUTP - OUT)).reshape(1, OUTP)

    sc12, sc3 = _sc_kernels()
    h1, st1s, st1d, r1 = _pre1(x, W1, att_src1, att_dst1, R1)
    acc1, s1, _ = sc12(src, dst, h1.reshape(N * H, C), st1s, st1d)
    h2, st2s, st2d, r2 = _mid(acc1, s1, r1, b1r, g1r, be1r,
                              W2, att_src2, att_dst2, R2, H, C, D)
    acc2, s2, _ = sc12(src, dst, h2.reshape(N * H, C), st2s, st2d)
    h3, st3s, st3d, r3 = _mid(acc2, s2, r2, b2r, g2r, be2r,
                              W3p, as3p, ad3p, R3p, 1, OUTP, OUTP)
    acc3, s3, _ = sc3(src, dst, h3.reshape(N * 2, C), st3s, st3d)
    out = _post3(acc3, s3, r3, b3p)
    return out[:, :OUT]


# speedup vs baseline: 1.0763x; 1.0763x over previous
import functools

import jax
import jax.numpy as jnp
from jax import lax
from jax.experimental import pallas as pl
from jax.experimental.pallas import tpu as pltpu
from jax.experimental.pallas import tpu_sc as plsc

N = 10000
E = 160000
IN_DIM = 50
H = 8
C = 64
D = 512
OUT = 121
OUTP = 128

BN = 1000
NB = N // BN

NCORES = 2
NSUB = 16
CH = 400
STRIPE = 624
TAIL = N - NSUB * STRIPE
ZROWS = 104

_f32 = jnp.float32
_i32 = jnp.int32


def _stats_from_h(h, as_ref, ad_ref, heads, c):
    bn = h.shape[0]
    cols_s, cols_d = [], []
    for h2 in range(heads):
        hh = h[:, c * h2:c * h2 + c]
        cols_s.append(jnp.sum(hh * as_ref[h2:h2 + 1, :], axis=1, keepdims=True))
        cols_d.append(jnp.sum(hh * ad_ref[h2:h2 + 1, :], axis=1, keepdims=True))
    pad = [jnp.zeros((bn, 16 - heads), dtype=_f32)]
    return (jnp.concatenate(cols_s + pad, axis=1),
            jnp.concatenate(cols_d + pad, axis=1))


def _pre1_body(x_ref, W1_ref, as_ref, ad_ref, R1_ref,
               h_ref, sts_ref, std_ref, r_ref):
    xb = x_ref[...]
    h = jnp.dot(xb, W1_ref[...], preferred_element_type=_f32)
    h_ref[...] = h
    sts_ref[...], std_ref[...] = _stats_from_h(h, as_ref, ad_ref, H, C)
    r_ref[...] = jnp.dot(xb, R1_ref[...], preferred_element_type=_f32)


def _epilogue(acc_ref, s_ref, r_ref, b_ref, g_ref, be_ref):
    s = s_ref[0, :, :8] + s_ref[1, :, :8]
    rinv = 1.0 / (s + 1e-16)
    acc = acc_ref[...]
    v = jnp.concatenate([acc[h2] * rinv[:, h2:h2 + 1] for h2 in range(H)],
                        axis=1)
    v = v + b_ref[...] + r_ref[...]
    mu = jnp.mean(v, axis=1, keepdims=True)
    var = jnp.mean((v - mu) ** 2, axis=1, keepdims=True)
    y = (v - mu) * lax.rsqrt(var + 1e-5) * g_ref[...] + be_ref[...]
    return jnp.where(y > 0, y, jnp.exp(jnp.minimum(y, 0.0)) - 1.0)


def _mid_body(W_ref, as_ref, ad_ref, R_ref,
              acc_ref, s_ref, r_ref, b_ref, g_ref, be_ref,
              h_ref, sts_ref, std_ref, r2_ref, *, heads, c):
    y = _epilogue(acc_ref, s_ref, r_ref, b_ref, g_ref, be_ref)
    h = jnp.dot(y, W_ref[...], preferred_element_type=_f32)
    h_ref[...] = h
    sts_ref[...], std_ref[...] = _stats_from_h(h, as_ref, ad_ref, heads, c)
    r2_ref[...] = jnp.dot(y, R_ref[...], preferred_element_type=_f32)


def _post3_body(acc_ref, s_ref, r_ref, b_ref, o_ref):
    o = jnp.concatenate([acc_ref[0] + acc_ref[2],
                         acc_ref[1] + acc_ref[3]], axis=1)
    stot = s_ref[0, :, 0:1] + s_ref[1, :, 0:1]
    o_ref[...] = o / (stot + 1e-16) + b_ref[...] + r_ref[...]


def _pre1(x, W1, as1, ad1, R1):
    full = lambda s: pl.BlockSpec(s, lambda i: (0,) * len(s))
    return pl.pallas_call(
        _pre1_body,
        grid=(NB,),
        in_specs=[pl.BlockSpec((BN, IN_DIM), lambda i: (i, 0)),
                  full((IN_DIM, D)), full((H, C)), full((H, C)),
                  full((IN_DIM, D))],
        out_specs=[pl.BlockSpec((BN, D), lambda i: (i, 0)),
                   pl.BlockSpec((BN, 16), lambda i: (i, 0)),
                   pl.BlockSpec((BN, 16), lambda i: (i, 0)),
                   pl.BlockSpec((BN, D), lambda i: (i, 0))],
        out_shape=[jax.ShapeDtypeStruct((N, D), _f32),
                   jax.ShapeDtypeStruct((N, 16), _f32),
                   jax.ShapeDtypeStruct((N, 16), _f32),
                   jax.ShapeDtypeStruct((N, D), _f32)],
    )(x, W1, as1, ad1, R1)


def _mid(acc, s, r, b2d, g2d, be2d, W, asx, adx, R, heads, c, dout):
    full = lambda sh: pl.BlockSpec(sh, lambda i: (0,) * len(sh))
    kdim = W.shape[0]
    return pl.pallas_call(
        functools.partial(_mid_body, heads=heads, c=c),
        grid=(NB,),
        in_specs=[full((kdim, dout)), full((heads, c)), full((heads, c)),
                  full((kdim, dout)),
                  pl.BlockSpec((H, BN, C), lambda i: (0, i, 0)),
                  pl.BlockSpec((2, BN, 16), lambda i: (0, i, 0)),
                  pl.BlockSpec((BN, D), lambda i: (i, 0)),
                  full((1, D)), full((1, D)), full((1, D))],
        out_specs=[pl.BlockSpec((BN, dout), lambda i: (i, 0)),
                   pl.BlockSpec((BN, 16), lambda i: (i, 0)),
                   pl.BlockSpec((BN, 16), lambda i: (i, 0)),
                   pl.BlockSpec((BN, dout), lambda i: (i, 0))],
        out_shape=[jax.ShapeDtypeStruct((N, dout), _f32),
                   jax.ShapeDtypeStruct((N, 16), _f32),
                   jax.ShapeDtypeStruct((N, 16), _f32),
                   jax.ShapeDtypeStruct((N, dout), _f32)],
    )(W, asx, adx, R, acc, s, r, b2d, g2d, be2d)


def _post3(acc3, s3, r3, b3p2d):
    full = lambda sh: pl.BlockSpec(sh, lambda i: (0,) * len(sh))
    return pl.pallas_call(
        _post3_body,
        grid=(NB,),
        in_specs=[pl.BlockSpec((4, BN, C), lambda i: (0, i, 0)),
                  pl.BlockSpec((2, BN, 16), lambda i: (0, i, 0)),
                  pl.BlockSpec((BN, OUTP), lambda i: (i, 0)),
                  full((1, OUTP))],
        out_specs=pl.BlockSpec((BN, OUTP), lambda i: (i, 0)),
        out_shape=jax.ShapeDtypeStruct((N, OUTP), _f32),
    )(acc3, s3, r3, b3p2d)


_COL8 = None


def _zero_vmem(ref, nrows, ncol):
    def row(i, _):
        for q in range(ncol // 16):
            ref[i, pl.ds(16 * q, 16)] = jnp.zeros((16,), _f32)
        return _
    lax.fori_loop(0, nrows, row, None)


def _for_stripe(sid, fn):
    fn(sid * STRIPE, STRIPE)

    @pl.when(sid == NSUB - 1)
    def _():
        fn(NSUB * STRIPE, TAIL)


def _zero_stripe(zb, acc, base, size):
    for k in range(size // ZROWS):
        pltpu.sync_copy(zb, acc.at[pl.ds(base + k * ZROWS, ZROWS)])
    rem = size % ZROWS
    if rem:
        pltpu.sync_copy(zb.at[pl.ds(0, rem)],
                        acc.at[pl.ds(base + size - rem, rem)])


def _compute_ex(sst, dstst, exb, n_edges):
    @plsc.parallel_loop(0, n_edges, unroll=8)
    def body(e):
        t = sst[e] + dstst[e]
        exb[e] = jnp.exp(jnp.maximum(t, 0.2 * t))


def _splat16(vec, lane):
    return vec.at[jnp.full((16,), lane, _i32)].get(mode="promise_in_bounds")


def _scale_rows(rows, exb, head, n_edges, nq):
    @plsc.parallel_loop(0, n_edges, unroll=8)
    def body(e):
        exs = _splat16(exb[e], head)
        for q in range(nq):
            rows[e, pl.ds(16 * q, 16)] = rows[e, pl.ds(16 * q, 16)] * exs


_NCHUNK12 = E // NSUB // CH


def _sc12_body(src_hbm, dst_hbm, hview, stats_s, stats_d,
               out_hm, s_out, ex_hbm,
               src_v, dst_v, idx_v, sst, dstst, exb, rows,
               zb, zbs, acc, s_acc, sem1, sem2, sem3):
    cid = lax.axis_index("c")
    sid = lax.axis_index("s")
    base0 = sid * (E // NSUB)

    for p in range(4):
        head = 4 * cid + p
        _zero_vmem(zb, ZROWS, C)
        _for_stripe(sid, functools.partial(_zero_stripe, zb, acc))
        if p == 0:
            _zero_vmem(zbs, ZROWS, 16)
            _for_stripe(sid, functools.partial(_zero_stripe, zbs, s_acc))
        plsc.subcore_barrier()

        def chunk(kc, _):
            eb = base0 + kc * CH
            pltpu.sync_copy(src_hbm.at[pl.ds(eb, CH)], src_v)
            pltpu.sync_copy(dst_hbm.at[pl.ds(eb, CH)], dst_v)
            for i in range(CH // 16):
                sl = pl.ds(16 * i, 16)
                idx_v[sl] = src_v[sl] * 8 + head
            gr = pltpu.async_copy(hview.at[idx_v], rows, sem3)
            if p == 0:
                ga = pltpu.async_copy(stats_s.at[src_v], sst, sem1)
                gb = pltpu.async_copy(stats_d.at[dst_v], dstst, sem2)
                ga.wait()
                gb.wait()
                _compute_ex(sst, dstst, exb, CH)
                pltpu.sync_copy(exb, ex_hbm.at[pl.ds(eb, CH)])
                mine = jnp.where(cid == 0, kc < 13, kc >= 13)

                @pl.when(mine)
                def _():
                    pltpu.sync_copy(exb, s_acc.at[dst_v], add=True)
            else:
                pltpu.sync_copy(ex_hbm.at[pl.ds(eb, CH)], exb)
            gr.wait()
            _scale_rows(rows, exb, head, CH, C // 16)
            pltpu.sync_copy(rows, acc.at[dst_v], add=True)
            return _

        lax.fori_loop(0, _NCHUNK12, chunk, None)
        plsc.subcore_barrier()

        def cp(base, size):
            sl = pl.ds(base, size)
            pltpu.sync_copy(acc.at[sl], out_hm.at[head, sl])
            if p == 0:
                pltpu.sync_copy(s_acc.at[sl], s_out.at[cid, sl])
        _for_stripe(sid, cp)


def _sc3_body(src_hbm, dst_hbm, hview, stats_s, stats_d,
              out_p, s_out, ex_hbm,
              src_v, dst_v, idx_v, sst, dstst, exb, rows,
              zb, zbs, acc, s_acc, sem1, sem2, sem3):
    cid = lax.axis_index("c")
    sid = lax.axis_index("s")
    base0 = sid * (E // NSUB)

    for q in range(2):
        _zero_vmem(zb, ZROWS, C)
        _for_stripe(sid, functools.partial(_zero_stripe, zb, acc))
        if q == 0:
            _zero_vmem(zbs, ZROWS, 16)
            _for_stripe(sid, functools.partial(_zero_stripe, zbs, s_acc))
        plsc.subcore_barrier()

        def chunk(kc, _):
            mine = jnp.where(cid == 0, kc < 13, kc >= 13)

            @pl.when(mine)
            def _run():
                eb = base0 + kc * CH
                pltpu.sync_copy(src_hbm.at[pl.ds(eb, CH)], src_v)
                pltpu.sync_copy(dst_hbm.at[pl.ds(eb, CH)], dst_v)
                if q == 0:
                    ga = pltpu.async_copy(stats_s.at[src_v], sst, sem1)
                    gb = pltpu.async_copy(stats_d.at[dst_v], dstst, sem2)
                    ga.wait()
                    gb.wait()
                    _compute_ex(sst, dstst, exb, CH)
                    pltpu.sync_copy(exb, ex_hbm.at[pl.ds(eb, CH)])
                    pltpu.sync_copy(exb, s_acc.at[dst_v], add=True)
                else:
                    pltpu.sync_copy(ex_hbm.at[pl.ds(eb, CH)], exb)
                for i in range(CH // 16):
                    sl = pl.ds(16 * i, 16)
                    idx_v[sl] = src_v[sl] * 2 + q
                pltpu.async_copy(hview.at[idx_v], rows, sem3).wait()
                _scale_rows(rows, exb, 0, CH, C // 16)
                pltpu.sync_copy(rows, acc.at[dst_v], add=True)
            return _

        lax.fori_loop(0, _NCHUNK12, chunk, None)
        plsc.subcore_barrier()

        def cp(base, size):
            sl = pl.ds(base, size)
            pltpu.sync_copy(acc.at[sl], out_p.at[2 * cid + q, sl])
            if q == 0:
                pltpu.sync_copy(s_acc.at[sl], s_out.at[cid, sl])
        _for_stripe(sid, cp)


@functools.cache
def _sc_kernels():
    mesh = plsc.VectorSubcoreMesh(core_axis_name="c", subcore_axis_name="s",
                                  num_cores=NCORES, num_subcores=NSUB)
    params = pltpu.CompilerParams(use_tc_tiling_on_sc=False)
    sc12 = pl.kernel(
        _sc12_body,
        out_type=[jax.ShapeDtypeStruct((H, N, C), _f32),
                  jax.ShapeDtypeStruct((2, N, 16), _f32),
                  jax.ShapeDtypeStruct((E, 16), _f32)],
        mesh=mesh,
        scratch_types=[
            pltpu.VMEM((CH,), _i32), pltpu.VMEM((CH,), _i32),
            pltpu.VMEM((CH,), _i32),
            pltpu.VMEM((CH, 16), _f32), pltpu.VMEM((CH, 16), _f32),
            pltpu.VMEM((CH, 16), _f32), pltpu.VMEM((CH, C), _f32),
            pltpu.VMEM((ZROWS, C), _f32), pltpu.VMEM((ZROWS, 16), _f32),
            pltpu.VMEM_SHARED((N, C), _f32), pltpu.VMEM_SHARED((N, 16), _f32),
            pltpu.SemaphoreType.DMA, pltpu.SemaphoreType.DMA,
            pltpu.SemaphoreType.DMA,
        ],
        compiler_params=params,
    )

    sc3 = pl.kernel(
        _sc3_body,
        out_type=[jax.ShapeDtypeStruct((4, N, C), _f32),
                  jax.ShapeDtypeStruct((2, N, 16), _f32),
                  jax.ShapeDtypeStruct((E, 16), _f32)],
        mesh=mesh,
        scratch_types=[
            pltpu.VMEM((CH,), _i32), pltpu.VMEM((CH,), _i32),
            pltpu.VMEM((CH,), _i32),
            pltpu.VMEM((CH, 16), _f32), pltpu.VMEM((CH, 16), _f32),
            pltpu.VMEM((CH, 16), _f32), pltpu.VMEM((CH, C), _f32),
            pltpu.VMEM((ZROWS, C), _f32), pltpu.VMEM((ZROWS, 16), _f32),
            pltpu.VMEM_SHARED((N, C), _f32), pltpu.VMEM_SHARED((N, 16), _f32),
            pltpu.SemaphoreType.DMA, pltpu.SemaphoreType.DMA,
            pltpu.SemaphoreType.DMA,
        ],
        compiler_params=params,
    )
    return sc12, sc3


def kernel(x, edge_index, W1, att_src1, att_dst1, b1, R1, g1, beta1,
           W2, att_src2, att_dst2, b2, R2, g2, beta2,
           W3, att_src3, att_dst3, b3, R3):
    src = edge_index[0].astype(_i32)
    dst = edge_index[1].astype(_i32)

    b1r, g1r, be1r = b1.reshape(1, D), g1.reshape(1, D), beta1.reshape(1, D)
    b2r, g2r, be2r = b2.reshape(1, D), g2.reshape(1, D), beta2.reshape(1, D)
    W3p = jnp.pad(W3, ((0, 0), (0, OUTP - OUT)))
    R3p = jnp.pad(R3, ((0, 0), (0, OUTP - OUT)))
    as3p = jnp.pad(att_src3, ((0, 0), (0, OUTP - OUT)))
    ad3p = jnp.pad(att_dst3, ((0, 0), (0, OUTP - OUT)))
    b3p = jnp.pad(b3, (0, OUTP - OUT)).reshape(1, OUTP)

    sc12, sc3 = _sc_kernels()
    h1, st1s, st1d, r1 = _pre1(x, W1, att_src1, att_dst1, R1)
    acc1, s1, _ = sc12(src, dst, h1.reshape(N * H, C), st1s, st1d)
    h2, st2s, st2d, r2 = _mid(acc1, s1, r1, b1r, g1r, be1r,
                              W2, att_src2, att_dst2, R2, H, C, D)
    acc2, s2, _ = sc12(src, dst, h2.reshape(N * H, C), st2s, st2d)
    h3, st3s, st3d, r3 = _mid(acc2, s2, r2, b2r, g2r, be2r,
                              W3p, as3p, ad3p, R3p, 1, OUTP, OUTP)
    acc3, s3, _ = sc3(src, dst, h3.reshape(N * 2, C), st3s, st3d)
    out = _post3(acc3, s3, r3, b3p)
    return out[:, :OUT]
